# baseline (speedup 1.0000x reference)
import functools

import jax
import jax.numpy as jnp
from jax.experimental import pallas as pl
from jax.experimental.pallas import tpu as pltpu

_NC = 2


def _moe_body(eids_ref, pairs_ref, wsort_ref, x_ref, w1_ref, w3_ref, w2_ref,
              out_ref, *, top_k):
    i = pl.program_id(0)
    c = pl.program_id(1)

    @pl.when((i == 0) & (c == 0))
    def _init():
        out_ref[...] = jnp.zeros_like(out_ref)

    p = pairs_ref[i]
    t = p // top_k
    xrow = x_ref[pl.ds(t, 1), :]
    g = jnp.dot(xrow, w1_ref[0], preferred_element_type=jnp.float32)
    u = jnp.dot(xrow, w3_ref[0], preferred_element_type=jnp.float32)
    h = (g * jax.nn.sigmoid(g)) * u
    o = jnp.dot(h, w2_ref[0], preferred_element_type=jnp.float32)
    w = wsort_ref[i]
    out_ref[pl.ds(t, 1), :] += w * o


def kernel(x, expert_indices, expert_weights, w1_stacked, w2_stacked, w3_stacked):
    B, H = x.shape
    K = expert_indices.shape[1]
    E, _, I = w1_stacked.shape
    P = B * K
    IC = I // _NC

    eids = expert_indices.reshape(P).astype(jnp.int32)
    order = jnp.argsort(eids).astype(jnp.int32)
    sorted_eids = eids[order]
    sorted_w = expert_weights.reshape(P)[order]

    grid_spec = pltpu.PrefetchScalarGridSpec(
        num_scalar_prefetch=3,
        grid=(P, _NC),
        in_specs=[
            pl.BlockSpec((B, H), lambda i, c, e, p, w: (0, 0)),
            pl.BlockSpec((1, H, IC), lambda i, c, e, p, w: (e[i], 0, c)),
            pl.BlockSpec((1, H, IC), lambda i, c, e, p, w: (e[i], 0, c)),
            pl.BlockSpec((1, IC, H), lambda i, c, e, p, w: (e[i], c, 0)),
        ],
        out_specs=pl.BlockSpec((B, H), lambda i, c, e, p, w: (0, 0)),
    )
    fn = pl.pallas_call(
        functools.partial(_moe_body, top_k=K),
        grid_spec=grid_spec,
        out_shape=jax.ShapeDtypeStruct((B, H), jnp.float32),
    )
    return fn(sorted_eids, order, sorted_w, x, w1_stacked, w3_stacked, w2_stacked)


# speedup vs baseline: 1.3149x; 1.3149x over previous
import functools

import jax
import jax.numpy as jnp
from jax.experimental import pallas as pl
from jax.experimental.pallas import tpu as pltpu


def _moe_body(eids_ref, pairs_ref, wsort_ref, x_ref, w1_ref, w3_ref, w2_ref,
              out_ref, *, top_k):
    i = pl.program_id(0)

    @pl.when(i == 0)
    def _init():
        out_ref[...] = jnp.zeros_like(out_ref)

    p = pairs_ref[i]
    t = p // top_k
    xrow = x_ref[pl.ds(t, 1), :]
    g = jnp.dot(xrow, w1_ref[0], preferred_element_type=jnp.float32)
    u = jnp.dot(xrow, w3_ref[0], preferred_element_type=jnp.float32)
    h = (g * jax.nn.sigmoid(g)) * u
    o = jnp.dot(h, w2_ref[0], preferred_element_type=jnp.float32)
    w = wsort_ref[i]
    out_ref[pl.ds(t, 1), :] += w * o


def _rank_sort(eids, wvals):
    P = eids.shape[0]
    ii = jnp.arange(P, dtype=jnp.int32)
    lt = eids[None, :] < eids[:, None]
    tie = (eids[None, :] == eids[:, None]) & (ii[None, :] < ii[:, None])
    rank = (lt | tie).sum(axis=1).astype(jnp.int32)
    onehot = (rank[None, :] == ii[:, None]).astype(jnp.float32)
    sorted_keys = (onehot @ eids.astype(jnp.float32)).astype(jnp.int32)
    order = (onehot @ ii.astype(jnp.float32)).astype(jnp.int32)
    sorted_w = onehot @ wvals
    return sorted_keys, order, sorted_w


def kernel(x, expert_indices, expert_weights, w1_stacked, w2_stacked, w3_stacked):
    B, H = x.shape
    K = expert_indices.shape[1]
    E, _, I = w1_stacked.shape
    P = B * K

    eids = expert_indices.reshape(P).astype(jnp.int32)
    sorted_eids, order, sorted_w = _rank_sort(eids, expert_weights.reshape(P))

    grid_spec = pltpu.PrefetchScalarGridSpec(
        num_scalar_prefetch=3,
        grid=(P,),
        in_specs=[
            pl.BlockSpec((B, H), lambda i, e, p, w: (0, 0)),
            pl.BlockSpec((1, H, I), lambda i, e, p, w: (e[i], 0, 0)),
            pl.BlockSpec((1, H, I), lambda i, e, p, w: (e[i], 0, 0)),
            pl.BlockSpec((1, I, H), lambda i, e, p, w: (e[i], 0, 0)),
        ],
        out_specs=pl.BlockSpec((B, H), lambda i, e, p, w: (0, 0)),
    )
    fn = pl.pallas_call(
        functools.partial(_moe_body, top_k=K),
        grid_spec=grid_spec,
        out_shape=jax.ShapeDtypeStruct((B, H), jnp.float32),
    )
    return fn(sorted_eids, order, sorted_w, x, w1_stacked, w3_stacked, w2_stacked)


# speedup vs baseline: 1.3166x; 1.0013x over previous
import functools

import jax
import jax.numpy as jnp
from jax.experimental import pallas as pl
from jax.experimental.pallas import tpu as pltpu


def _moe_body(eids_ref, pairs_ref, wsort_ref, x_ref, w1_ref, w3_ref, w2_ref,
              out_ref, *, top_k):
    i = pl.program_id(0)

    @pl.when(i == 0)
    def _init():
        out_ref[...] = jnp.zeros_like(out_ref)

    p = pairs_ref[i]
    t = p // top_k
    xrow = x_ref[pl.ds(t, 1), :]
    g = jnp.dot(xrow, w1_ref[0], preferred_element_type=jnp.float32)
    u = jnp.dot(xrow, w3_ref[0], preferred_element_type=jnp.float32)
    h = (g * jax.nn.sigmoid(g)) * u
    o = jnp.dot(h, w2_ref[0], preferred_element_type=jnp.float32)
    w = wsort_ref[i]
    out_ref[pl.ds(t, 1), :] += w * o


def _rank_sort(eids, wvals):
    P = eids.shape[0]
    ii = jnp.arange(P, dtype=jnp.int32)
    lt = eids[None, :] < eids[:, None]
    tie = (eids[None, :] == eids[:, None]) & (ii[None, :] < ii[:, None])
    rank = (lt | tie).sum(axis=1).astype(jnp.int32)
    onehot = rank[None, :] == ii[:, None]
    sorted_keys = jnp.where(onehot, eids[None, :], 0).sum(axis=1)
    order = jnp.where(onehot, ii[None, :], 0).sum(axis=1)
    sorted_w = wvals[order]
    return sorted_keys.astype(jnp.int32), order.astype(jnp.int32), sorted_w


def kernel(x, expert_indices, expert_weights, w1_stacked, w2_stacked, w3_stacked):
    B, H = x.shape
    K = expert_indices.shape[1]
    E, _, I = w1_stacked.shape
    P = B * K

    eids = expert_indices.reshape(P).astype(jnp.int32)
    sorted_eids, order, sorted_w = _rank_sort(eids, expert_weights.reshape(P))

    grid_spec = pltpu.PrefetchScalarGridSpec(
        num_scalar_prefetch=3,
        grid=(P,),
        in_specs=[
            pl.BlockSpec((B, H), lambda i, e, p, w: (0, 0)),
            pl.BlockSpec((1, H, I), lambda i, e, p, w: (e[i], 0, 0)),
            pl.BlockSpec((1, H, I), lambda i, e, p, w: (e[i], 0, 0)),
            pl.BlockSpec((1, I, H), lambda i, e, p, w: (e[i], 0, 0)),
        ],
        out_specs=pl.BlockSpec((B, H), lambda i, e, p, w: (0, 0)),
    )
    fn = pl.pallas_call(
        functools.partial(_moe_body, top_k=K),
        grid_spec=grid_spec,
        out_shape=jax.ShapeDtypeStruct((B, H), jnp.float32),
    )
    return fn(sorted_eids, order, sorted_w, x, w1_stacked, w3_stacked, w2_stacked)


# speedup vs baseline: 1.6268x; 1.2356x over previous
import functools

import jax
import jax.numpy as jnp
from jax.experimental import pallas as pl
from jax.experimental.pallas import tpu as pltpu

_NBUF = 3


def _moe_body(uexp_ref, starts_ref, cnts_ref, pairs_ref, wsort_ref, d_ref,
              x_ref, w1_any, w3_any, w2_any, out_ref,
              w1b, w3b, w2b, sems, *, top_k):
    d = d_ref[0]
    out_ref[...] = jnp.zeros_like(out_ref)

    def _copies(j, slot):
        e = uexp_ref[j]
        return (
            pltpu.make_async_copy(w1_any.at[e], w1b.at[slot], sems.at[slot, 0]),
            pltpu.make_async_copy(w3_any.at[e], w3b.at[slot], sems.at[slot, 1]),
            pltpu.make_async_copy(w2_any.at[e], w2b.at[slot], sems.at[slot, 2]),
        )

    for jj in range(_NBUF):
        @pl.when(jj < d)
        def _start():
            for c in _copies(jj, jj):
                c.start()

    def seg_body(j, carry):
        slot = jax.lax.rem(j, _NBUF)
        for c in _copies(j, slot):
            c.wait()

        start = starts_ref[j]
        cnt = cnts_ref[j]

        def row_body(r, carry2):
            p = pairs_ref[r]
            t = p // top_k
            xrow = x_ref[pl.ds(t, 1), :]
            g = jnp.dot(xrow, w1b[slot], preferred_element_type=jnp.float32)
            u = jnp.dot(xrow, w3b[slot], preferred_element_type=jnp.float32)
            h = (g * jax.nn.sigmoid(g)) * u
            o = jnp.dot(h, w2b[slot], preferred_element_type=jnp.float32)
            out_ref[pl.ds(t, 1), :] += wsort_ref[r] * o
            return carry2

        jax.lax.fori_loop(start, start + cnt, row_body, 0)

        @pl.when(j + _NBUF < d)
        def _next():
            for c in _copies(j + _NBUF, slot):
                c.start()
        return carry

    jax.lax.fori_loop(0, d, seg_body, 0)


def _route(eids, wvals):
    P = eids.shape[0]
    ii = jnp.arange(P, dtype=jnp.int32)
    lt = eids[None, :] < eids[:, None]
    tie = (eids[None, :] == eids[:, None]) & (ii[None, :] < ii[:, None])
    rank = (lt | tie).sum(axis=1).astype(jnp.int32)
    onehot = rank[None, :] == ii[:, None]
    sorted_eids = jnp.where(onehot, eids[None, :], 0).sum(axis=1)
    order = jnp.where(onehot, ii[None, :], 0).sum(axis=1).astype(jnp.int32)
    sorted_w = wvals[order]

    new = jnp.concatenate([jnp.ones((1,), jnp.int32),
                           (sorted_eids[1:] != sorted_eids[:-1]).astype(jnp.int32)])
    segid = jnp.cumsum(new) - 1
    d = new.sum().astype(jnp.int32)
    seg_onehot = segid[None, :] == ii[:, None]
    is_first = seg_onehot & (new[None, :] == 1)
    uexp = jnp.where(is_first, sorted_eids[None, :], 0).sum(axis=1).astype(jnp.int32)
    starts = jnp.where(is_first, ii[None, :], 0).sum(axis=1).astype(jnp.int32)
    cnts = seg_onehot.sum(axis=1).astype(jnp.int32)
    return uexp, starts, cnts, order, sorted_w, d


def kernel(x, expert_indices, expert_weights, w1_stacked, w2_stacked, w3_stacked):
    B, H = x.shape
    K = expert_indices.shape[1]
    E, _, I = w1_stacked.shape
    P = B * K

    eids = expert_indices.reshape(P).astype(jnp.int32)
    uexp, starts, cnts, order, sorted_w, d = _route(eids, expert_weights.reshape(P))
    darr = d.reshape(1)

    grid_spec = pltpu.PrefetchScalarGridSpec(
        num_scalar_prefetch=6,
        grid=(1,),
        in_specs=[
            pl.BlockSpec((B, H), lambda i, *_: (0, 0)),
            pl.BlockSpec(memory_space=pl.ANY),
            pl.BlockSpec(memory_space=pl.ANY),
            pl.BlockSpec(memory_space=pl.ANY),
        ],
        out_specs=pl.BlockSpec((B, H), lambda i, *_: (0, 0)),
        scratch_shapes=[
            pltpu.VMEM((_NBUF, H, I), jnp.float32),
            pltpu.VMEM((_NBUF, H, I), jnp.float32),
            pltpu.VMEM((_NBUF, I, H), jnp.float32),
            pltpu.SemaphoreType.DMA((_NBUF, 3)),
        ],
    )
    fn = pl.pallas_call(
        functools.partial(_moe_body, top_k=K),
        grid_spec=grid_spec,
        out_shape=jax.ShapeDtypeStruct((B, H), jnp.float32),
    )
    return fn(uexp, starts, cnts, order, sorted_w, darr,
              x, w1_stacked, w3_stacked, w2_stacked)
